# baseline (speedup 1.0000x reference)
import functools

import numpy as np

import jax
import jax.numpy as jnp
from jax import lax
from jax.experimental import pallas as pl
from jax.experimental.pallas import tpu as pltpu
from jax.experimental.pallas import tpu_sc as plsc

NHID = 128
N_NODES = 10000
N_EDGES = 320000

NC = 2
NS = 16
NW = NC * NS
K = 112
NCH = 90
BC = 6
NB = NCH // BC
EPW = NCH * K
E_PAD = NW * EPW
N_PAD = 10240
ROWS_PER_TILE = N_PAD // NS


def _sc_aggregate(h, src3, dst3, zeros):
    mesh = plsc.VectorSubcoreMesh(core_axis_name="c", subcore_axis_name="s")

    @functools.partial(
        pl.kernel,
        mesh=mesh,
        out_type=jax.ShapeDtypeStruct((NC, N_PAD, NHID), jnp.float32),
        scratch_types=[
            pltpu.VMEM((2, BC, K), jnp.int32),
            pltpu.VMEM((2, BC, K), jnp.int32),
            pltpu.VMEM((K, NHID), jnp.float32),
            pltpu.VMEM((K, NHID), jnp.float32),
            pltpu.VMEM((K, NHID), jnp.float32),
            pltpu.VMEM_SHARED((N_PAD, NHID), jnp.float32),
            pltpu.SemaphoreType.DMA,
            pltpu.SemaphoreType.DMA,
            pltpu.SemaphoreType.DMA,
            pltpu.SemaphoreType.DMA,
            pltpu.SemaphoreType.DMA,
            pltpu.SemaphoreType.DMA,
        ],
    )
    def agg_kernel(h_hbm, src_hbm, dst_hbm, z_hbm, out_hbm,
                   src_v, dst_v, rows_a, rows_b, rows_c, acc,
                   ga, gb, gc, i0, i1, zs):
        c = lax.axis_index("c")
        s = lax.axis_index("s")
        w = s * NC + c
        isems = (i0, i1)
        rbufs = (rows_a, rows_b, rows_c)
        gsems = (ga, gb, gc)
        stripe = pl.ds(s * ROWS_PER_TILE, ROWS_PER_TILE)

        def idx_issue(blk, ring):
            pltpu.async_copy(src_hbm.at[w, blk], src_v.at[ring], isems[ring])
            pltpu.async_copy(dst_hbm.at[w, blk], dst_v.at[ring], isems[ring])

        def idx_wait(blk, ring):
            pltpu.make_async_copy(
                src_hbm.at[w, blk], src_v.at[ring], isems[ring]).wait()
            pltpu.make_async_copy(
                dst_hbm.at[w, blk], dst_v.at[ring], isems[ring]).wait()

        def gather_issue(ring, cc, buf):
            pltpu.async_copy(
                h_hbm.at[src_v.at[ring, cc]], rbufs[buf], gsems[buf])

        def gather_wait(buf):
            pltpu.make_async_copy(
                h_hbm.at[src_v.at[0, 0]], rbufs[buf], gsems[buf]).wait()

        init_copy = pltpu.async_copy(z_hbm.at[stripe], acc.at[stripe], zs)
        idx_issue(0, 0)
        idx_issue(1, 1)
        idx_wait(0, 0)
        gather_issue(0, 0, 0)
        gather_issue(0, 1, 1)
        gather_issue(0, 2, 2)
        init_copy.wait()
        plsc.subcore_barrier()

        def block(b, ring, do_prefetch, do_next):
            for cc in range(BC):
                buf = cc % 3
                gather_wait(buf)
                pltpu.sync_copy(rbufs[buf], acc.at[dst_v.at[ring, cc]], add=True)
                if cc + 3 < BC:
                    gather_issue(ring, cc + 3, buf)
                elif do_next:
                    if cc + 3 == BC:
                        idx_wait(b + 1, 1 - ring)
                    gather_issue(1 - ring, cc + 3 - BC, buf)
            if do_prefetch:
                idx_issue(b + 2, ring)

        @pl.loop(0, NB - 3, step=2)
        def _(bb):
            block(bb, 0, True, True)
            block(bb + 1, 1, True, True)

        block(NB - 3, 0, True, True)
        block(NB - 2, 1, False, True)
        block(NB - 1, 0, False, False)

        plsc.subcore_barrier()
        pltpu.sync_copy(acc.at[stripe], out_hbm.at[c, stripe])

    return agg_kernel(h, src3, dst3, zeros)


def _tc_mlp(h, p, W1, b1, W2, b2):
    BLK = 2000

    def body(h_ref, p_ref, w1_ref, b1_ref, w2_ref, b2_ref, o_ref):
        z = h_ref[...] + p_ref[0] + p_ref[1]
        z = jnp.dot(z.astype(jnp.bfloat16), w1_ref[...].astype(jnp.bfloat16),
                    preferred_element_type=jnp.float32)
        z = jnp.maximum(z + b1_ref[...], 0.0)
        o_ref[...] = (
            jnp.dot(z.astype(jnp.bfloat16), w2_ref[...].astype(jnp.bfloat16),
                    preferred_element_type=jnp.float32)
            + b2_ref[...]
        )

    return pl.pallas_call(
        body,
        grid=(N_NODES // BLK,),
        in_specs=[
            pl.BlockSpec((BLK, NHID), lambda i: (i, 0)),
            pl.BlockSpec((NC, BLK, NHID), lambda i: (0, i, 0)),
            pl.BlockSpec((NHID, NHID), lambda i: (0, 0)),
            pl.BlockSpec((1, NHID), lambda i: (0, 0)),
            pl.BlockSpec((NHID, NHID), lambda i: (0, 0)),
            pl.BlockSpec((1, NHID), lambda i: (0, 0)),
        ],
        out_specs=pl.BlockSpec((BLK, NHID), lambda i: (i, 0)),
        out_shape=jax.ShapeDtypeStruct((N_NODES, NHID), jnp.float32),
    )(h, p, W1, b1.reshape(1, NHID), W2, b2.reshape(1, NHID))


def kernel(x, edge_index, W1_0, b1_0, W2_0, b2_0, W1_1, b1_1, W2_1, b2_1):
    ppt = EPW - N_EDGES // NW
    tix = np.arange(NW, dtype=np.int32)[:, None]
    pix = np.arange(ppt, dtype=np.int32)[None, :]
    pad_src = jnp.asarray((tix * 313 + pix * 37) % N_NODES)
    pad_dst = jnp.asarray(N_NODES + (tix * 7 + pix) % (N_PAD - N_NODES))
    src3 = jnp.concatenate(
        [edge_index[0].astype(jnp.int32).reshape(NW, N_EDGES // NW),
         pad_src], axis=1).reshape(NW, NB, BC, K)
    dst3 = jnp.concatenate(
        [edge_index[1].astype(jnp.int32).reshape(NW, N_EDGES // NW),
         pad_dst], axis=1).reshape(NW, NB, BC, K)
    zeros = jnp.zeros((N_PAD, NHID), jnp.float32)
    h = x
    for (W1, b1, W2, b2) in ((W1_0, b1_0, W2_0, b2_0), (W1_1, b1_1, W2_1, b2_1)):
        p = _sc_aggregate(h, src3, dst3, zeros)
        h = _tc_mlp(h, p, W1, b1, W2, b2)
    return h


# speedup vs baseline: 1.0063x; 1.0063x over previous
import functools

import numpy as np

import jax
import jax.numpy as jnp
from jax import lax
from jax.experimental import pallas as pl
from jax.experimental.pallas import tpu as pltpu
from jax.experimental.pallas import tpu_sc as plsc

NHID = 128
N_NODES = 10000
N_EDGES = 320000

NC = 2
NS = 16
NW = NC * NS
K = 112
NCH = 90
BC = 6
NB = NCH // BC
EPW = NCH * K
E_PAD = NW * EPW
N_PAD = 10240
ROWS_PER_TILE = N_PAD // NS


def _sc_aggregate(h, src3, dst3, zeros):
    mesh = plsc.VectorSubcoreMesh(core_axis_name="c", subcore_axis_name="s")

    @functools.partial(
        pl.kernel,
        mesh=mesh,
        out_type=jax.ShapeDtypeStruct((NC, N_PAD, NHID), jnp.float32),
        scratch_types=[
            pltpu.VMEM((2, BC, K), jnp.int32),
            pltpu.VMEM((2, BC, K), jnp.int32),
            pltpu.VMEM((K, NHID), jnp.float32),
            pltpu.VMEM((K, NHID), jnp.float32),
            pltpu.VMEM((K, NHID), jnp.float32),
            pltpu.VMEM_SHARED((N_PAD, NHID), jnp.float32),
            pltpu.SemaphoreType.DMA,
            pltpu.SemaphoreType.DMA,
            pltpu.SemaphoreType.DMA,
            pltpu.SemaphoreType.DMA,
            pltpu.SemaphoreType.DMA,
            pltpu.SemaphoreType.DMA,
        ],
    )
    def agg_kernel(h_hbm, src_hbm, dst_hbm, z_hbm, out_hbm,
                   src_v, dst_v, rows_a, rows_b, rows_c, acc,
                   ga, gb, gc, i0, i1, zs):
        c = lax.axis_index("c")
        s = lax.axis_index("s")
        w = s * NC + c
        isems = (i0, i1)
        rbufs = (rows_a, rows_b, rows_c)
        gsems = (ga, gb, gc)
        stripe = pl.ds(s * ROWS_PER_TILE, ROWS_PER_TILE)

        def idx_issue(blk, ring):
            pltpu.async_copy(src_hbm.at[w, blk], src_v.at[ring], isems[ring])
            pltpu.async_copy(dst_hbm.at[w, blk], dst_v.at[ring], isems[ring])

        def idx_wait(blk, ring):
            pltpu.make_async_copy(
                src_hbm.at[w, blk], src_v.at[ring], isems[ring]).wait()
            pltpu.make_async_copy(
                dst_hbm.at[w, blk], dst_v.at[ring], isems[ring]).wait()

        def gather_issue(ring, cc, buf):
            pltpu.async_copy(
                h_hbm.at[src_v.at[ring, cc]], rbufs[buf], gsems[buf])

        def gather_wait(buf):
            pltpu.make_async_copy(
                h_hbm.at[src_v.at[0, 0]], rbufs[buf], gsems[buf]).wait()

        init_copy = pltpu.async_copy(z_hbm.at[stripe], acc.at[stripe], zs)
        idx_issue(0, 0)
        idx_issue(1, 1)
        idx_wait(0, 0)
        gather_issue(0, 0, 0)
        gather_issue(0, 1, 1)
        gather_issue(0, 2, 2)
        init_copy.wait()
        plsc.subcore_barrier()

        def block(b, ring, do_prefetch, do_next):
            for cc in range(BC):
                buf = cc % 3
                gather_wait(buf)
                pltpu.sync_copy(rbufs[buf], acc.at[dst_v.at[ring, cc]], add=True)
                if cc + 3 < BC:
                    gather_issue(ring, cc + 3, buf)
                elif do_next:
                    if cc + 3 == BC:
                        idx_wait(b + 1, 1 - ring)
                    gather_issue(1 - ring, cc + 3 - BC, buf)
            if do_prefetch:
                idx_issue(b + 2, ring)

        @pl.loop(0, NB - 3, step=2)
        def _(bb):
            block(bb, 0, True, True)
            block(bb + 1, 1, True, True)

        block(NB - 3, 0, True, True)
        block(NB - 2, 1, False, True)
        block(NB - 1, 0, False, False)

        plsc.subcore_barrier()
        pltpu.sync_copy(acc.at[stripe], out_hbm.at[c, stripe])

    return agg_kernel(h, src3, dst3, zeros)


def _tc_mlp(h, p, W1, b1, W2, b2):
    BLK = 2000

    def body(h_ref, p_ref, w1_ref, b1_ref, w2_ref, b2_ref, o_ref):
        z = h_ref[...] + p_ref[0] + p_ref[1]
        z = jnp.dot(z, w1_ref[...], preferred_element_type=jnp.float32)
        z = jnp.maximum(z + b1_ref[...], 0.0)
        o_ref[...] = (
            jnp.dot(z, w2_ref[...], preferred_element_type=jnp.float32)
            + b2_ref[...]
        )

    return pl.pallas_call(
        body,
        grid=(N_NODES // BLK,),
        in_specs=[
            pl.BlockSpec((BLK, NHID), lambda i: (i, 0)),
            pl.BlockSpec((NC, BLK, NHID), lambda i: (0, i, 0)),
            pl.BlockSpec((NHID, NHID), lambda i: (0, 0)),
            pl.BlockSpec((1, NHID), lambda i: (0, 0)),
            pl.BlockSpec((NHID, NHID), lambda i: (0, 0)),
            pl.BlockSpec((1, NHID), lambda i: (0, 0)),
        ],
        out_specs=pl.BlockSpec((BLK, NHID), lambda i: (i, 0)),
        out_shape=jax.ShapeDtypeStruct((N_NODES, NHID), jnp.float32),
    )(h, p, W1, b1.reshape(1, NHID), W2, b2.reshape(1, NHID))


def kernel(x, edge_index, W1_0, b1_0, W2_0, b2_0, W1_1, b1_1, W2_1, b2_1):
    ppt = EPW - N_EDGES // NW
    tix = np.arange(NW, dtype=np.int32)[:, None]
    pix = np.arange(ppt, dtype=np.int32)[None, :]
    pad_src = jnp.asarray((tix * 313 + pix * 37) % N_NODES)
    pad_dst = jnp.asarray(N_NODES + (tix * 7 + pix) % (N_PAD - N_NODES))
    src3 = jnp.concatenate(
        [edge_index[0].astype(jnp.int32).reshape(NW, N_EDGES // NW),
         pad_src], axis=1).reshape(NW, NB, BC, K)
    dst3 = jnp.concatenate(
        [edge_index[1].astype(jnp.int32).reshape(NW, N_EDGES // NW),
         pad_dst], axis=1).reshape(NW, NB, BC, K)
    zeros = jnp.zeros((N_PAD, NHID), jnp.float32)
    h = x
    for (W1, b1, W2, b2) in ((W1_0, b1_0, W2_0, b2_0), (W1_1, b1_1, W2_1, b2_1)):
        p = _sc_aggregate(h, src3, dst3, zeros)
        h = _tc_mlp(h, p, W1, b1, W2, b2)
    return h
